# baseline (speedup 1.0000x reference)
import functools

import jax
import jax.numpy as jnp
from jax import lax
from jax.experimental import pallas as pl
from jax.experimental.pallas import tpu as pltpu
from jax.experimental.pallas import tpu_sc as plsc

Z = 32
ROW = 2 * Z
NC, NS = 2, 16
NW = NC * NS
SUB = 128
NSUB = 5
CHUNK = SUB * NSUB
NSLOT = 3
NBUF = 2

_MESH = dict(core_axis_name="c", subcore_axis_name="s")
_NO_TC_TILING = pltpu.CompilerParams(use_tc_tiling_on_sc=False)


def _make_widen_kernel(n_rows):
    cols = 512
    grid = (n_rows + cols - 1) // cols

    def body(in_ref, out_ref):
        out_ref[:, 0:ROW] = in_ref[...].T

    return pl.pallas_call(
        body,
        grid=(grid,),
        in_specs=[pl.BlockSpec((ROW, cols), lambda i: (0, i))],
        out_specs=pl.BlockSpec((cols, 128), lambda i: (i, 0)),
        out_shape=jax.ShapeDtypeStruct((n_rows, 128), jnp.float32),
    )


def _transform3d(src_ref, dst_ref, l, base):
    @plsc.parallel_loop(0, CHUNK, unroll=4)
    def body(ii):
        for h in range(Z // 16):
            m = src_ref[ii, pl.ds(h * 16, 16)]
            lv = src_ref[ii, pl.ds(Z + h * 16, 16)]
            i = base + ii
            dst_ref[i // l, i % l, pl.ds(h * 16, 16)] = jnp.exp(lv * 0.5) + m


def _make_stream_kernel(b, l):
    total = b * l
    per_w = total // NW
    n_chunks = per_w // CHUNK
    b_chunk = CHUNK // l
    lpad = ((l + 7) // 8) * 8
    mesh = plsc.VectorSubcoreMesh(**_MESH)

    @functools.partial(
        pl.kernel,
        mesh=mesh,
        compiler_params=_NO_TC_TILING,
        out_type=jax.ShapeDtypeStruct((b * lpad, 128), jnp.float32),
        scratch_types=[
            pltpu.VMEM((NSUB, SUB), jnp.int32),
            pltpu.VMEM((CHUNK, 128), jnp.float32),
            pltpu.VMEM((NBUF, b_chunk, l, Z), jnp.float32),
        ] + [pltpu.SemaphoreType.DMA] * (1 + NBUF),
    )
    def k(idx_hbm, tab_hbm, out_hbm, idx_v, ebuf_v, obuf_v, *sems):
        gsem = sems[0]
        osems = sems[1:]
        wid = lax.axis_index("s") * NC + lax.axis_index("c")
        idx_base = wid * (per_w // SUB)
        out_base = wid * (per_w // l) * lpad

        pending_o = [None] * NBUF

        for c in range(n_chunks):
            p = c % NBUF
            pltpu.sync_copy(idx_hbm.at[pl.ds(idx_base + c * NSUB, NSUB)],
                            idx_v)
            gathers = [
                pltpu.async_copy(tab_hbm.at[idx_v.at[j]],
                                 ebuf_v.at[pl.ds(j * SUB, SUB)], gsem)
                for j in range(NSUB)
            ]
            if pending_o[p] is not None:
                for cp in pending_o[p]:
                    cp.wait()
                pending_o[p] = None
            for cp in gathers:
                cp.wait()
            _transform3d(ebuf_v, obuf_v.at[p], l, 0)
            pending_o[p] = [
                pltpu.async_copy(
                    obuf_v.at[p, bb],
                    out_hbm.at[pl.ds(out_base + (c * b_chunk + bb) * lpad, l),
                               pl.ds(0, Z)],
                    osems[p])
                for bb in range(b_chunk)
            ]
        for po in pending_o:
            if po is not None:
                for cp in po:
                    cp.wait()

    return k


def kernel(s, r, o, e_table, r_table):
    b, l = s.shape
    total = b * l

    s_idx = s.reshape(total // SUB, SUB).astype(jnp.int32)
    o_idx = o.reshape(total // SUB, SUB).astype(jnp.int32)
    r_idx = r.reshape(total // SUB, SUB).astype(jnp.int32)

    lpad = ((l + 7) // 8) * 8
    n_e = e_table.shape[0]
    nr = r_table.shape[0]
    e_wide = _make_widen_kernel(n_e)(e_table.T)
    r_wide = _make_widen_kernel(nr)(r_table.T)

    gk = _make_stream_kernel(b, l)

    def run(idx, tab):
        out2 = gk(idx, tab)
        return out2.reshape(b, lpad, 128)[:, :l, :Z]

    zs = run(s_idx, e_wide)
    zo = run(o_idx, e_wide)
    zr = run(r_idx, r_wide)
    return (zs, zr, zo)


# speedup vs baseline: 1.7882x; 1.7882x over previous
import functools

import jax
import jax.numpy as jnp
from jax import lax
from jax.experimental import pallas as pl
from jax.experimental.pallas import tpu as pltpu
from jax.experimental.pallas import tpu_sc as plsc

Z = 32
ROW = 2 * Z
NC, NS = 2, 16
NW = NC * NS
SUB = 128
NSUB = 5
CHUNK = SUB * NSUB
NBUF = 2

_MESH = dict(core_axis_name="c", subcore_axis_name="s")
_NO_TC_TILING = pltpu.CompilerParams(use_tc_tiling_on_sc=False)


def _transform3d(src_ref, dst_ref, l):
    @plsc.parallel_loop(0, CHUNK, unroll=4)
    def body(i):
        for h in range(Z // 16):
            m = src_ref[i, pl.ds(h * 16, 16)]
            lv = src_ref[i, pl.ds(Z + h * 16, 16)]
            dst_ref[i // l, i % l, pl.ds(h * 16, 16)] = jnp.exp(lv * 0.5) + m


def _make_stream_kernel(b, l):
    total = b * l
    per_w = total // NW
    n_chunks = per_w // CHUNK
    b_chunk = CHUNK // l
    lpad = ((l + 7) // 8) * 8
    mesh = plsc.VectorSubcoreMesh(**_MESH)

    @functools.partial(
        pl.kernel,
        mesh=mesh,
        compiler_params=_NO_TC_TILING,
        out_type=jax.ShapeDtypeStruct((b * lpad, 128), jnp.float32),
        scratch_types=[
            pltpu.VMEM((NBUF, NSUB, SUB), jnp.int32),
            pltpu.VMEM((NBUF, CHUNK, ROW), jnp.float32),
            pltpu.VMEM((NBUF, b_chunk, l, Z), jnp.float32),
        ] + [pltpu.SemaphoreType.DMA] * (2 * NBUF),
    )
    def k(idx_hbm, tab_hbm, out_hbm, idx_v, ebuf_v, obuf_v, *sems):
        gsems = sems[:NBUF]
        osems = sems[NBUF:]
        wid = lax.axis_index("s") * NC + lax.axis_index("c")
        idx_base = wid * (per_w // SUB)
        out_base = wid * (per_w // l) * lpad

        pending_g = [None] * NBUF
        pending_o = [None] * NBUF

        def prep(c):
            p = c % NBUF
            if pending_o[p] is not None:
                for cp in pending_o[p]:
                    cp.wait()
                pending_o[p] = None
            pltpu.sync_copy(idx_hbm.at[pl.ds(idx_base + c * NSUB, NSUB)],
                            idx_v.at[p])
            pending_g[p] = [
                pltpu.async_copy(tab_hbm.at[idx_v.at[p, j]],
                                 ebuf_v.at[p, pl.ds(j * SUB, SUB)], gsems[p])
                for j in range(NSUB)
            ]

        def complete(c):
            p = c % NBUF
            for cp in pending_g[p]:
                cp.wait()
            pending_g[p] = None
            _transform3d(ebuf_v.at[p], obuf_v.at[p], l)
            pending_o[p] = [
                pltpu.async_copy(
                    obuf_v.at[p, bb],
                    out_hbm.at[pl.ds(out_base + (c * b_chunk + bb) * lpad, l),
                               pl.ds(0, Z)],
                    osems[p])
                for bb in range(b_chunk)
            ]

        depth = NBUF - 1
        for c in range(min(depth, n_chunks)):
            prep(c)
        for c in range(n_chunks):
            if c + depth < n_chunks:
                prep(c + depth)
            complete(c)
        for po in pending_o:
            if po is not None:
                for cp in po:
                    cp.wait()

    return k


def kernel(s, r, o, e_table, r_table):
    b, l = s.shape
    total = b * l

    s_idx = s.reshape(total // SUB, SUB).astype(jnp.int32)
    o_idx = o.reshape(total // SUB, SUB).astype(jnp.int32)
    r_idx = r.reshape(total // SUB, SUB).astype(jnp.int32)

    lpad = ((l + 7) // 8) * 8
    gk = _make_stream_kernel(b, l)

    def run(idx, tab):
        out2 = gk(idx, tab)
        return out2.reshape(b, lpad, 128)[:, :l, :Z]

    zs = run(s_idx, e_table)
    zo = run(o_idx, e_table)
    zr = run(r_idx, r_table)
    return (zs, zr, zo)
